# baseline (speedup 1.0000x reference)
import jax
import jax.numpy as jnp
from jax import lax
from jax.experimental import pallas as pl
from jax.experimental.pallas import tpu as pltpu
from jax.experimental.pallas import tpu_sc as plsc

_P = 81
_B = 16384
_NW = 32
_TQ = 32
_NU = 10


def _gather_slab(tblv, inb, st, r, tq):
    ch = 0 if r < 2 else 1

    @plsc.parallel_loop(0, tq, unroll=2)
    def tchunk(tt):
        idx = [inb[tt, ch, pl.ds(16 * g, 16)] for g in range(8)]
        for s in range(8):
            row = tblv.at[pl.ds((r * 8 + s) * 16, 16)]
            for g in range(8):
                st[tt, s, pl.ds(16 * g, 16)] = plsc.load_gather(
                    row, [idx[g]])


def _body(img_ref, tbl_ref, out_ref, tblv, inbuf, stage, sem_in, sem_out):
    wid = lax.axis_index("s") * 2 + lax.axis_index("c")
    pltpu.sync_copy(tbl_ref, tblv)

    def in_copy(ui, ib):
        u = wid + _NW * ui
        p = u // 4
        t0 = _TQ * (u - 4 * p)
        return (pltpu.make_async_copy(
            img_ref.at[p, pl.ds(t0, _TQ)], inbuf.at[ib], sem_in), p, t0)

    in_copy(0, 0)[0].start()

    def unit(ui, _):
        ib = ui % 2
        cp, p, t0 = in_copy(ui, ib)
        cp.wait()

        @pl.when(ui + 1 < _NU)
        def _prefetch():
            in_copy(ui + 1, 1 - ib)[0].start()

        for r in range(4):
            sb = r % 2
            out_slab = out_ref.at[p, r, pl.ds(t0, _TQ)]
            ring_wait = pltpu.make_async_copy(stage.at[sb], out_slab, sem_out)
            if r < 2:
                @pl.when(ui > 0)
                def _drain():
                    ring_wait.wait()
            else:
                ring_wait.wait()
            _gather_slab(tblv, inbuf.at[ib], stage.at[sb], r, _TQ)
            pltpu.make_async_copy(stage.at[sb], out_slab, sem_out).start()
        return ()

    lax.fori_loop(0, _NU, unit, ())
    for sb in range(2):
        pltpu.make_async_copy(
            stage.at[sb], out_ref.at[0, 0, pl.ds(0, _TQ)], sem_out).wait()

    t0 = 4 * wid
    pltpu.sync_copy(img_ref.at[_P - 1, pl.ds(t0, 4)],
                    inbuf.at[0, pl.ds(0, 4)])
    for r in range(4):
        _gather_slab(tblv, inbuf.at[0], stage.at[0], r, 4)
        pltpu.sync_copy(stage.at[0, pl.ds(0, 4)],
                        out_ref.at[_P - 1, r, pl.ds(t0, 4)])


@jax.jit
def _encode(img_lin, tbl):
    mesh = plsc.VectorSubcoreMesh(core_axis_name="c", subcore_axis_name="s")
    return pl.kernel(
        _body,
        out_type=jax.ShapeDtypeStruct((_P, 4, 128, 8, 128), jnp.float32),
        mesh=mesh,
        scratch_types=[
            pltpu.VMEM((512,), jnp.float32),
            pltpu.VMEM((2, _TQ, 2, 128), jnp.int32),
            pltpu.VMEM((2, _TQ, 8, 128), jnp.float32),
            pltpu.SemaphoreType.DMA,
            pltpu.SemaphoreType.DMA,
        ],
        compiler_params=pltpu.CompilerParams(
            needs_layout_passes=False, use_tc_tiling_on_sc=False),
    )(img_lin, tbl)


def kernel(img, entity_table, color_table):
    tbl = jnp.concatenate(
        [entity_table.T, jnp.pad(color_table.T, ((0, 0), (0, 4)))],
        axis=0).reshape(-1)
    img_lin = jnp.transpose(img, (1, 2, 0, 3)).reshape(9, 9, 128, 128, 2)
    img_lin = jnp.transpose(img_lin, (0, 1, 2, 4, 3)).reshape(_P, 128, 2, 128)
    out_lin = _encode(img_lin, tbl)
    return jnp.transpose(out_lin.reshape(9, 9, 4, 128, 8, 128),
                         (3, 5, 0, 1, 2, 4)).reshape(_B, 9, 9, 32)
